# baseline (speedup 1.0000x reference)
import jax
import jax.numpy as jnp
from jax import lax
from jax.experimental import pallas as pl
from jax.experimental.pallas import tpu as pltpu
from jax.experimental.pallas import tpu_sc as plsc

N = 10000
E = 320000
D = 128
NC = 2
NS = 16
NW = NC * NS
EPW = E // NW
B = 48
NCHB = 208
TB = 16
NVR = D // 16
RPT = 624
ZR = 8


def _feats_body(x_ref, pos_ref, wl_ref, wr_ref, we_ref, att_ref,
                u_ref, v_ref, xl_ref):
    x = x_ref[...]
    p = pos_ref[...] @ we_ref[...]
    xl = x @ wl_ref[...]
    xr = x @ wr_ref[...]
    a6 = 0.6 * att_ref[...]
    u_ref[...] = (xl - p) * a6
    v_ref[...] = (xr + p) * a6
    xl_ref[...] = xl


def _feats(x, pos, W_l, W_r, W_e, att):
    bn = 2000
    grid = N // bn
    return pl.pallas_call(
        _feats_body,
        grid=(grid,),
        in_specs=[
            pl.BlockSpec((bn, D), lambda i: (i, 0)),
            pl.BlockSpec((bn, 3), lambda i: (i, 0)),
            pl.BlockSpec((D, D), lambda i: (0, 0)),
            pl.BlockSpec((D, D), lambda i: (0, 0)),
            pl.BlockSpec((3, D), lambda i: (0, 0)),
            pl.BlockSpec((1, D), lambda i: (0, 0)),
        ],
        out_specs=[
            pl.BlockSpec((bn, D), lambda i: (i, 0)),
            pl.BlockSpec((bn, D), lambda i: (i, 0)),
            pl.BlockSpec((bn, D), lambda i: (i, 0)),
        ],
        out_shape=[
            jax.ShapeDtypeStruct((N, D), jnp.float32),
            jax.ShapeDtypeStruct((N, D), jnp.float32),
            jax.ShapeDtypeStruct((N, D), jnp.float32),
        ],
    )(x, pos, W_l, W_r, W_e, att)


def _edges_body(u_h, v_h, xl_h, att_h, sdm_h, sdt_h,
                outp_h, denp_h,
                idx_v, tidx_v, att_v, urows, vrows, xlrows,
                ex_v, exr_v, zrows, zvec, out_sh, den_sh,
                sem_g, sem_s, sem_i):
    c = lax.axis_index("c")
    s = lax.axis_index("s")
    wid = c * NS + s

    pltpu.sync_copy(att_h, att_v)

    zero16 = jnp.zeros((16,), jnp.float32)

    def _zrow_body(i, _):
        r = i // NVR
        j = i % NVR
        zrows[r, pl.ds(j * 16, 16)] = zero16
        return 0

    lax.fori_loop(0, ZR * NVR, _zrow_body, 0)

    def _zvec_body(i, _):
        zvec[pl.ds(i * 16, 16)] = zero16
        return 0

    lax.fori_loop(0, 1000 // 16, _zvec_body, 0)

    for i in range(RPT // ZR):
        pltpu.sync_copy(zrows, out_sh.at[pl.ds(s * RPT + i * ZR, ZR)])

    @pl.when(s == 0)
    def _():
        for i in range((N - NS * RPT) // ZR):
            pltpu.sync_copy(zrows, out_sh.at[pl.ds(NS * RPT + i * ZR, ZR)])
        for i in range(N // 1000):
            pltpu.sync_copy(zvec, den_sh.at[pl.ds(i * 1000, 1000)])

    plsc.subcore_barrier()

    cpos = jnp.full((16,), 2.0 / 3.0, jnp.float32)
    sg_regs = [jnp.where(att_v[pl.ds(j * 16, 16)] > 0, cpos, -cpos)
               for j in range(NVR)]
    lane0 = lax.iota(jnp.int32, 16)

    def lane_sum(acc):
        for kk in (8, 4, 2, 1):
            idx = (lane0 + kk) & 15
            acc = acc + jnp.take_along_axis(acc, idx, axis=0,
                                            mode="promise_in_bounds")
        return acc

    def idx_load(k):
        pltpu.async_copy(sdm_h.at[wid, k], idx_v.at[k & 3], sem_i)

    def idx_drain():
        pltpu.make_async_copy(sdm_h.at[0, 0], idx_v.at[0], sem_i).wait()

    def gather_issue(k):
        q = k & 3
        p = k & 1
        pltpu.async_copy(u_h.at[idx_v.at[q, 0]], urows.at[p], sem_g)
        pltpu.async_copy(v_h.at[idx_v.at[q, 1]], vrows.at[p], sem_g)
        pltpu.async_copy(xl_h.at[idx_v.at[q, 0]], xlrows.at[p], sem_g)

    def gather_drain():
        for _ in range(3):
            pltpu.make_async_copy(u_h.at[pl.ds(0, B)], urows.at[0],
                                  sem_g).wait()

    def scatter_issue(k):
        q = k & 3
        p = k & 1
        pltpu.async_copy(ex_v.at[p], den_sh.at[idx_v.at[q, 1]], sem_s,
                         add=True)
        pltpu.async_copy(xlrows.at[p], out_sh.at[idx_v.at[q, 1]], sem_s,
                         add=True)

    def scatter_drain(k):
        q = k & 3
        p = k & 1
        pltpu.make_async_copy(ex_v.at[p], den_sh.at[idx_v.at[q, 1]],
                              sem_s).wait()
        pltpu.make_async_copy(xlrows.at[p], out_sh.at[idx_v.at[q, 1]],
                              sem_s).wait()

    def compute(pt, ngrp):
        def grp_body(g, _):
            for l in range(16):
                e = g * 16 + l
                acc_a = zero16
                acc_b = zero16
                for j in range(NVR):
                    sl = pl.ds(j * 16, 16)
                    t = urows[pt, e, sl] + vrows[pt, e, sl]
                    w = t + sg_regs[j] * jnp.abs(t)
                    if j % 2 == 0:
                        acc_a = acc_a + w
                    else:
                        acc_b = acc_b + w
                exv = jnp.exp(lane_sum(acc_a + acc_b))
                exr_v[pl.ds(e * 16, 16)] = exv
                for j in range(NVR):
                    sl = pl.ds(j * 16, 16)
                    xlrows[pt, e, sl] = xlrows[pt, e, sl] * exv
            ex_v[pt, pl.ds(g * 16, 16)] = plsc.load_gather(
                exr_v, [(g * 16 + lane0) * 16])
            return 0

        lax.fori_loop(0, ngrp, grp_body, 0)

    idx_load(0)
    idx_drain()
    gather_issue(0)
    idx_load(1)

    gather_drain()
    idx_drain()
    idx_load(2)
    gather_issue(1)
    compute(0, B // 16)
    scatter_issue(0)

    def chunk_body(k, _):
        p = k & 1
        gather_drain()
        scatter_drain(k - 1)
        idx_drain()
        idx_load(k + 2)
        gather_issue(k + 1)
        compute(p, B // 16)
        scatter_issue(k)
        return 0

    lax.fori_loop(1, NCHB - 2, chunk_body, 0)

    gather_drain()
    scatter_drain(NCHB - 3)
    idx_drain()
    gather_issue(NCHB - 1)
    compute((NCHB - 2) & 1, B // 16)
    scatter_issue(NCHB - 2)

    gather_drain()
    scatter_drain(NCHB - 2)
    compute((NCHB - 1) & 1, B // 16)
    scatter_issue(NCHB - 1)
    scatter_drain(NCHB - 1)

    pltpu.sync_copy(sdt_h.at[wid], tidx_v)
    cp_u = pltpu.async_copy(u_h.at[tidx_v.at[0]], urows.at[0, pl.ds(0, TB)],
                            sem_g)
    cp_v = pltpu.async_copy(v_h.at[tidx_v.at[1]], vrows.at[0, pl.ds(0, TB)],
                            sem_g)
    cp_x = pltpu.async_copy(xl_h.at[tidx_v.at[0]], xlrows.at[0, pl.ds(0, TB)],
                            sem_g)
    cp_u.wait()
    cp_v.wait()
    cp_x.wait()
    compute(0, TB // 16)
    pltpu.sync_copy(ex_v.at[0, pl.ds(0, TB)], den_sh.at[tidx_v.at[1]],
                    add=True)
    pltpu.sync_copy(xlrows.at[0, pl.ds(0, TB)], out_sh.at[tidx_v.at[1]],
                    add=True)

    plsc.subcore_barrier()

    pltpu.sync_copy(out_sh.at[pl.ds(s * RPT, RPT)],
                    outp_h.at[c, pl.ds(s * RPT, RPT)])

    @pl.when(s == 0)
    def _():
        pltpu.sync_copy(den_sh, denp_h.at[c])
        pltpu.sync_copy(out_sh.at[pl.ds(NS * RPT, N - NS * RPT)],
                        outp_h.at[c, pl.ds(NS * RPT, N - NS * RPT)])


def _edges(u, v, xl, att1d, sdm, sdt):
    mesh = plsc.VectorSubcoreMesh(core_axis_name="c", subcore_axis_name="s")
    f = pl.kernel(
        _edges_body,
        out_type=[
            jax.ShapeDtypeStruct((NC, N, D), jnp.float32),
            jax.ShapeDtypeStruct((NC, N), jnp.float32),
        ],
        mesh=mesh,
        compiler_params=pltpu.CompilerParams(needs_layout_passes=False),
        scratch_types=[
            pltpu.VMEM((4, 2, B), jnp.int32),
            pltpu.VMEM((2, TB), jnp.int32),
            pltpu.VMEM((D,), jnp.float32),
            pltpu.VMEM((2, B, D), jnp.float32),
            pltpu.VMEM((2, B, D), jnp.float32),
            pltpu.VMEM((2, B, D), jnp.float32),
            pltpu.VMEM((2, B), jnp.float32),
            pltpu.VMEM((B * 16,), jnp.float32),
            pltpu.VMEM((ZR, D), jnp.float32),
            pltpu.VMEM((1000,), jnp.float32),
            pltpu.VMEM_SHARED((N, D), jnp.float32),
            pltpu.VMEM_SHARED((N,), jnp.float32),
            pltpu.SemaphoreType.DMA,
            pltpu.SemaphoreType.DMA,
            pltpu.SemaphoreType.DMA,
        ],
    )
    return f(u, v, xl, att1d, sdm, sdt)


def _finalize_body(outp_ref, denp_ref, gamma_ref, beta_ref, o_ref):
    p = outp_ref[0] + outp_ref[1]
    dsum = (denp_ref[0] + denp_ref[1] + 1e-16).reshape(N, 1)
    out = p / dsum
    mean = jnp.mean(out, axis=0, keepdims=True)
    var = jnp.mean((out - mean) ** 2, axis=0, keepdims=True)
    o_ref[...] = (out - mean) / jnp.sqrt(var + 1e-5) * gamma_ref[...] + beta_ref[...]


def _finalize(outp, denp, gamma, beta):
    return pl.pallas_call(
        _finalize_body,
        out_shape=jax.ShapeDtypeStruct((N, D), jnp.float32),
    )(outp, denp.reshape(NC, N, 1), gamma.reshape(1, D), beta.reshape(1, D))


def kernel(x, pos, edge_index, W_l, W_r, W_e, att, gamma, beta):
    src2 = edge_index[0].reshape(NW, EPW)
    dst2 = edge_index[1].reshape(NW, EPW)
    sdm = jnp.stack([src2[:, :NCHB * B].reshape(NW, NCHB, B),
                     dst2[:, :NCHB * B].reshape(NW, NCHB, B)],
                    axis=2)
    sdt = jnp.stack([src2[:, NCHB * B:], dst2[:, NCHB * B:]],
                    axis=1)
    u, v, xl = _feats(x, pos, W_l, W_r, W_e, att.reshape(1, D))
    outp, denp = _edges(u, v, xl, att.reshape(D), sdm, sdt)
    return _finalize(outp, denp, gamma, beta)


# speedup vs baseline: 1.0774x; 1.0774x over previous
import jax
import jax.numpy as jnp
from jax import lax
from jax.experimental import pallas as pl
from jax.experimental.pallas import tpu as pltpu
from jax.experimental.pallas import tpu_sc as plsc

N = 10000
E = 320000
D = 128
NC = 2
NS = 16
NW = NC * NS
EPW = E // NW
B = 48
NCHB = 208
TB = 16
NVR = D // 16
RPT = 624
ZR = 8


def _feats_body(x_ref, pos_ref, wl_ref, wr_ref, we_ref, att_ref,
                u_ref, v_ref, xl_ref):
    x = x_ref[...]
    p = pos_ref[...] @ we_ref[...]
    xl = x @ wl_ref[...]
    xr = x @ wr_ref[...]
    a6 = 0.6 * att_ref[...]
    u_ref[...] = (xl - p) * a6
    v_ref[...] = (xr + p) * a6
    xl_ref[...] = xl


def _feats(x, pos, W_l, W_r, W_e, att):
    bn = 2000
    grid = N // bn
    return pl.pallas_call(
        _feats_body,
        grid=(grid,),
        in_specs=[
            pl.BlockSpec((bn, D), lambda i: (i, 0)),
            pl.BlockSpec((bn, 3), lambda i: (i, 0)),
            pl.BlockSpec((D, D), lambda i: (0, 0)),
            pl.BlockSpec((D, D), lambda i: (0, 0)),
            pl.BlockSpec((3, D), lambda i: (0, 0)),
            pl.BlockSpec((1, D), lambda i: (0, 0)),
        ],
        out_specs=[
            pl.BlockSpec((bn, D), lambda i: (i, 0)),
            pl.BlockSpec((bn, D), lambda i: (i, 0)),
            pl.BlockSpec((bn, D), lambda i: (i, 0)),
        ],
        out_shape=[
            jax.ShapeDtypeStruct((N, D), jnp.float32),
            jax.ShapeDtypeStruct((N, D), jnp.float32),
            jax.ShapeDtypeStruct((N, D), jnp.float32),
        ],
    )(x, pos, W_l, W_r, W_e, att)


def _edges_body(u_h, v_h, xl_h, att_h, sdm_h, sdt_h,
                outp_h, denp_h,
                idx_v, tidx_v, att_v, urows, vrows, xlrows,
                ex_v, exr_v, zrows, zvec, out_sh, den_sh,
                sem_g, sem_s, sem_i):
    c = lax.axis_index("c")
    s = lax.axis_index("s")
    wid = c * NS + s

    pltpu.sync_copy(att_h, att_v)

    zero16 = jnp.zeros((16,), jnp.float32)

    def _zrow_body(i, _):
        r = i // NVR
        j = i % NVR
        zrows[r, pl.ds(j * 16, 16)] = zero16
        return 0

    lax.fori_loop(0, ZR * NVR, _zrow_body, 0)

    def _zvec_body(i, _):
        zvec[pl.ds(i * 16, 16)] = zero16
        return 0

    lax.fori_loop(0, 1000 // 16, _zvec_body, 0)

    for i in range(RPT // ZR):
        pltpu.sync_copy(zrows, out_sh.at[pl.ds(s * RPT + i * ZR, ZR)])

    @pl.when(s == 0)
    def _():
        for i in range((N - NS * RPT) // ZR):
            pltpu.sync_copy(zrows, out_sh.at[pl.ds(NS * RPT + i * ZR, ZR)])
        for i in range(N // 1000):
            pltpu.sync_copy(zvec, den_sh.at[pl.ds(i * 1000, 1000)])

    plsc.subcore_barrier()

    cpos = jnp.full((16,), 2.0 / 3.0, jnp.float32)
    sg_regs = [jnp.where(att_v[pl.ds(j * 16, 16)] > 0, cpos, -cpos)
               for j in range(NVR)]
    lane0 = lax.iota(jnp.int32, 16)

    def lane_sum(acc):
        for kk in (8, 4, 2, 1):
            idx = (lane0 + kk) & 15
            acc = acc + jnp.take_along_axis(acc, idx, axis=0,
                                            mode="promise_in_bounds")
        return acc

    def idx_load(k):
        pltpu.async_copy(sdm_h.at[wid, k], idx_v.at[k & 3], sem_i)

    def idx_drain():
        pltpu.make_async_copy(sdm_h.at[0, 0], idx_v.at[0], sem_i).wait()

    def gather_issue(k):
        q = k & 3
        p = k & 1
        pltpu.async_copy(u_h.at[idx_v.at[q, 0]], urows.at[p], sem_g)
        pltpu.async_copy(v_h.at[idx_v.at[q, 1]], vrows.at[p], sem_g)
        pltpu.async_copy(xl_h.at[idx_v.at[q, 0]], xlrows.at[p], sem_g)

    def gather_drain():
        for _ in range(3):
            pltpu.make_async_copy(u_h.at[pl.ds(0, B)], urows.at[0],
                                  sem_g).wait()

    def scatter_issue(k):
        q = k & 3
        p = k & 1
        pltpu.async_copy(ex_v.at[p], den_sh.at[idx_v.at[q, 1]], sem_s,
                         add=True)
        pltpu.async_copy(xlrows.at[p], out_sh.at[idx_v.at[q, 1]], sem_s,
                         add=True)

    def scatter_drain(k):
        q = k & 3
        p = k & 1
        pltpu.make_async_copy(ex_v.at[p], den_sh.at[idx_v.at[q, 1]],
                              sem_s).wait()
        pltpu.make_async_copy(xlrows.at[p], out_sh.at[idx_v.at[q, 1]],
                              sem_s).wait()

    def compute(pt, ngrp):
        def grp_body(g, _):
            for l in range(16):
                e = g * 16 + l
                acc_a = zero16
                acc_b = zero16
                for j in range(NVR):
                    sl = pl.ds(j * 16, 16)
                    t = urows[pt, e, sl] + vrows[pt, e, sl]
                    w = t + sg_regs[j] * jnp.abs(t)
                    if j % 2 == 0:
                        acc_a = acc_a + w
                    else:
                        acc_b = acc_b + w
                exr_v[pl.ds(e * 16, 16)] = lane_sum(acc_a + acc_b)
            ex16 = jnp.exp(plsc.load_gather(exr_v, [(g * 16 + lane0) * 16]))
            ex_v[pt, pl.ds(g * 16, 16)] = ex16
            for l in range(16):
                e = g * 16 + l
                sc = ex16[l]
                for j in range(NVR):
                    sl = pl.ds(j * 16, 16)
                    xlrows[pt, e, sl] = xlrows[pt, e, sl] * sc
            return 0

        lax.fori_loop(0, ngrp, grp_body, 0)

    idx_load(0)
    idx_drain()
    gather_issue(0)
    idx_load(1)

    gather_drain()
    idx_drain()
    idx_load(2)
    gather_issue(1)
    compute(0, B // 16)
    scatter_issue(0)

    def chunk_body(k, _):
        p = k & 1
        gather_drain()
        scatter_drain(k - 1)
        idx_drain()
        idx_load(k + 2)
        gather_issue(k + 1)
        compute(p, B // 16)
        scatter_issue(k)
        return 0

    lax.fori_loop(1, NCHB - 2, chunk_body, 0)

    gather_drain()
    scatter_drain(NCHB - 3)
    idx_drain()
    gather_issue(NCHB - 1)
    compute((NCHB - 2) & 1, B // 16)
    scatter_issue(NCHB - 2)

    gather_drain()
    scatter_drain(NCHB - 2)
    compute((NCHB - 1) & 1, B // 16)
    scatter_issue(NCHB - 1)
    scatter_drain(NCHB - 1)

    pltpu.sync_copy(sdt_h.at[wid], tidx_v)
    cp_u = pltpu.async_copy(u_h.at[tidx_v.at[0]], urows.at[0, pl.ds(0, TB)],
                            sem_g)
    cp_v = pltpu.async_copy(v_h.at[tidx_v.at[1]], vrows.at[0, pl.ds(0, TB)],
                            sem_g)
    cp_x = pltpu.async_copy(xl_h.at[tidx_v.at[0]], xlrows.at[0, pl.ds(0, TB)],
                            sem_g)
    cp_u.wait()
    cp_v.wait()
    cp_x.wait()
    compute(0, TB // 16)
    pltpu.sync_copy(ex_v.at[0, pl.ds(0, TB)], den_sh.at[tidx_v.at[1]],
                    add=True)
    pltpu.sync_copy(xlrows.at[0, pl.ds(0, TB)], out_sh.at[tidx_v.at[1]],
                    add=True)

    plsc.subcore_barrier()

    pltpu.sync_copy(out_sh.at[pl.ds(s * RPT, RPT)],
                    outp_h.at[c, pl.ds(s * RPT, RPT)])

    @pl.when(s == 0)
    def _():
        pltpu.sync_copy(den_sh, denp_h.at[c])
        pltpu.sync_copy(out_sh.at[pl.ds(NS * RPT, N - NS * RPT)],
                        outp_h.at[c, pl.ds(NS * RPT, N - NS * RPT)])


def _edges(u, v, xl, att1d, sdm, sdt):
    mesh = plsc.VectorSubcoreMesh(core_axis_name="c", subcore_axis_name="s")
    f = pl.kernel(
        _edges_body,
        out_type=[
            jax.ShapeDtypeStruct((NC, N, D), jnp.float32),
            jax.ShapeDtypeStruct((NC, N), jnp.float32),
        ],
        mesh=mesh,
        compiler_params=pltpu.CompilerParams(needs_layout_passes=False),
        scratch_types=[
            pltpu.VMEM((4, 2, B), jnp.int32),
            pltpu.VMEM((2, TB), jnp.int32),
            pltpu.VMEM((D,), jnp.float32),
            pltpu.VMEM((2, B, D), jnp.float32),
            pltpu.VMEM((2, B, D), jnp.float32),
            pltpu.VMEM((2, B, D), jnp.float32),
            pltpu.VMEM((2, B), jnp.float32),
            pltpu.VMEM((B * 16,), jnp.float32),
            pltpu.VMEM((ZR, D), jnp.float32),
            pltpu.VMEM((1000,), jnp.float32),
            pltpu.VMEM_SHARED((N, D), jnp.float32),
            pltpu.VMEM_SHARED((N,), jnp.float32),
            pltpu.SemaphoreType.DMA,
            pltpu.SemaphoreType.DMA,
            pltpu.SemaphoreType.DMA,
        ],
    )
    return f(u, v, xl, att1d, sdm, sdt)


def _finalize_body(outp_ref, denp_ref, gamma_ref, beta_ref, o_ref):
    p = outp_ref[0] + outp_ref[1]
    dsum = (denp_ref[0] + denp_ref[1] + 1e-16).reshape(N, 1)
    out = p / dsum
    mean = jnp.mean(out, axis=0, keepdims=True)
    var = jnp.mean((out - mean) ** 2, axis=0, keepdims=True)
    o_ref[...] = (out - mean) / jnp.sqrt(var + 1e-5) * gamma_ref[...] + beta_ref[...]


def _finalize(outp, denp, gamma, beta):
    return pl.pallas_call(
        _finalize_body,
        out_shape=jax.ShapeDtypeStruct((N, D), jnp.float32),
    )(outp, denp.reshape(NC, N, 1), gamma.reshape(1, D), beta.reshape(1, D))


def kernel(x, pos, edge_index, W_l, W_r, W_e, att, gamma, beta):
    src2 = edge_index[0].reshape(NW, EPW)
    dst2 = edge_index[1].reshape(NW, EPW)
    sdm = jnp.stack([src2[:, :NCHB * B].reshape(NW, NCHB, B),
                     dst2[:, :NCHB * B].reshape(NW, NCHB, B)],
                    axis=2)
    sdt = jnp.stack([src2[:, NCHB * B:], dst2[:, NCHB * B:]],
                    axis=1)
    u, v, xl = _feats(x, pos, W_l, W_r, W_e, att.reshape(1, D))
    outp, denp = _edges(u, v, xl, att.reshape(D), sdm, sdt)
    return _finalize(outp, denp, gamma, beta)


# speedup vs baseline: 1.8486x; 1.7157x over previous
import jax
import jax.numpy as jnp
from jax import lax
from jax.experimental import pallas as pl
from jax.experimental.pallas import tpu as pltpu
from jax.experimental.pallas import tpu_sc as plsc

N = 10000
E = 320000
D = 128
NC = 2
NS = 16
NW = NC * NS
EPW = E // NW
B = 80
NCH = 125
NVR = D // 16
RPT = 624
ZR = 8


def _feats_body(x_ref, pos_ref, wl_ref, wr_ref, we_ref, att_ref,
                u_ref, v_ref, xl_ref):
    x = x_ref[...]
    p = pos_ref[...] @ we_ref[...]
    xl = x @ wl_ref[...]
    xr = x @ wr_ref[...]
    a6 = 0.6 * att_ref[...]
    u_ref[...] = (xl - p) * a6
    v_ref[...] = (xr + p) * a6
    xl_ref[...] = xl


def _feats(x, pos, W_l, W_r, W_e, att):
    bn = 2000
    grid = N // bn
    return pl.pallas_call(
        _feats_body,
        grid=(grid,),
        in_specs=[
            pl.BlockSpec((bn, D), lambda i: (i, 0)),
            pl.BlockSpec((bn, 3), lambda i: (i, 0)),
            pl.BlockSpec((D, D), lambda i: (0, 0)),
            pl.BlockSpec((D, D), lambda i: (0, 0)),
            pl.BlockSpec((3, D), lambda i: (0, 0)),
            pl.BlockSpec((1, D), lambda i: (0, 0)),
        ],
        out_specs=[
            pl.BlockSpec((bn, D), lambda i: (i, 0)),
            pl.BlockSpec((bn, D), lambda i: (i, 0)),
            pl.BlockSpec((bn, D), lambda i: (i, 0)),
        ],
        out_shape=[
            jax.ShapeDtypeStruct((N, D), jnp.float32),
            jax.ShapeDtypeStruct((N, D), jnp.float32),
            jax.ShapeDtypeStruct((N, D), jnp.float32),
        ],
    )(x, pos, W_l, W_r, W_e, att)


def _edges_body(u_h, v_h, xl_h, att_h, sdm_h,
                outp_h, denp_h,
                idx_v, att_v, urows, vrows, xlrows,
                red_v, ex_v, zrows, zvec, out_sh, den_sh,
                sem_g, sem_x, sem_s, sem_i):
    c = lax.axis_index("c")
    s = lax.axis_index("s")
    wid = c * NS + s

    pltpu.sync_copy(att_h, att_v)

    zero16 = jnp.zeros((16,), jnp.float32)

    def _zrow_body(i, _):
        r = i // NVR
        j = i % NVR
        zrows[r, pl.ds(j * 16, 16)] = zero16
        return 0

    lax.fori_loop(0, ZR * NVR, _zrow_body, 0)

    def _zvec_body(i, _):
        zvec[pl.ds(i * 16, 16)] = zero16
        return 0

    lax.fori_loop(0, 1000 // 16, _zvec_body, 0)

    for i in range(RPT // ZR):
        pltpu.sync_copy(zrows, out_sh.at[pl.ds(s * RPT + i * ZR, ZR)])

    @pl.when(s == 0)
    def _():
        for i in range((N - NS * RPT) // ZR):
            pltpu.sync_copy(zrows, out_sh.at[pl.ds(NS * RPT + i * ZR, ZR)])
        for i in range(N // 1000):
            pltpu.sync_copy(zvec, den_sh.at[pl.ds(i * 1000, 1000)])

    plsc.subcore_barrier()

    cpos = jnp.full((16,), 2.0 / 3.0, jnp.float32)
    sg_regs = [jnp.where(att_v[pl.ds(j * 16, 16)] > 0, cpos, -cpos)
               for j in range(NVR)]
    lane0 = lax.iota(jnp.int32, 16)
    zero16i = jnp.zeros((16,), jnp.int32)

    def lane_sum(acc):
        for kk in (8, 4, 2, 1):
            idx = (lane0 + kk) & 15
            acc = acc + jnp.take_along_axis(acc, idx, axis=0,
                                            mode="promise_in_bounds")
        return acc

    def idx_load(k):
        pltpu.async_copy(sdm_h.at[wid, k], idx_v.at[k & 1], sem_i)

    def idx_drain():
        pltpu.make_async_copy(sdm_h.at[0, 0], idx_v.at[0], sem_i).wait()

    def iss_uv(k):
        q = k & 1
        pltpu.async_copy(u_h.at[idx_v.at[q, 0]], urows, sem_g)
        pltpu.async_copy(v_h.at[idx_v.at[q, 1]], vrows, sem_g)

    def drain_uv():
        for _ in range(2):
            pltpu.make_async_copy(u_h.at[pl.ds(0, B)], urows, sem_g).wait()

    def iss_xl(k):
        pltpu.async_copy(xl_h.at[idx_v.at[k & 1, 0]], xlrows, sem_x)

    def drain_xl():
        pltpu.make_async_copy(u_h.at[pl.ds(0, B)], xlrows, sem_x).wait()

    def iss_scatters(k):
        q = k & 1
        pltpu.async_copy(ex_v.at[q], den_sh.at[idx_v.at[q, 1]], sem_s,
                         add=True)
        pltpu.async_copy(xlrows, out_sh.at[idx_v.at[q, 1]], sem_s, add=True)

    def drain_scatters(k):
        q = k & 1
        pltpu.make_async_copy(ex_v.at[q], den_sh.at[idx_v.at[q, 1]],
                              sem_s).wait()
        pltpu.make_async_copy(xlrows, out_sh.at[idx_v.at[q, 1]],
                              sem_s).wait()

    def edge_loop():
        def edge_body(e, _):
            acc_a = zero16
            acc_b = zero16
            for j in range(NVR):
                sl = pl.ds(j * 16, 16)
                t = urows[e, sl] + vrows[e, sl]
                w = t + sg_regs[j] * jnp.abs(t)
                if j % 2 == 0:
                    acc_a = acc_a + w
                else:
                    acc_b = acc_b + w
            red_v[e, :] = lane_sum(acc_a + acc_b)
            return 0

        lax.fori_loop(0, B, edge_body, 0)

    def group_loop(k):
        q = k & 1

        def grp_body(g, _):
            e16 = g * 16 + lane0
            ex16 = jnp.exp(plsc.load_gather(red_v, [e16, zero16i]))
            ex_v[q, pl.ds(g * 16, 16)] = ex16
            return 0

        lax.fori_loop(0, B // 16, grp_body, 0)

    def scale_loop(k):
        q = k & 1

        def sc_body(g, _):
            ex16 = ex_v[q, pl.ds(g * 16, 16)]
            for l in range(16):
                e = g * 16 + l
                sc = ex16[l]
                for j in range(NVR):
                    sl = pl.ds(j * 16, 16)
                    xlrows[e, sl] = xlrows[e, sl] * sc
            return 0

        lax.fori_loop(0, B // 16, sc_body, 0)

    idx_load(0)

    idx_drain()
    iss_uv(0)
    iss_xl(0)
    idx_load(1)
    drain_uv()
    edge_loop()
    group_loop(0)
    drain_xl()
    scale_loop(0)
    iss_scatters(0)

    def chunk_body(k, _):
        idx_drain()
        iss_uv(k)
        drain_scatters(k - 1)
        iss_xl(k)
        idx_load(k + 1)
        drain_uv()
        edge_loop()
        group_loop(k)
        drain_xl()
        scale_loop(k)
        iss_scatters(k)
        return 0

    lax.fori_loop(1, NCH - 1, chunk_body, 0)

    idx_drain()
    iss_uv(NCH - 1)
    drain_scatters(NCH - 2)
    iss_xl(NCH - 1)
    drain_uv()
    edge_loop()
    group_loop(NCH - 1)
    drain_xl()
    scale_loop(NCH - 1)
    iss_scatters(NCH - 1)
    drain_scatters(NCH - 1)

    plsc.subcore_barrier()

    pltpu.sync_copy(out_sh.at[pl.ds(s * RPT, RPT)],
                    outp_h.at[c, pl.ds(s * RPT, RPT)])

    @pl.when(s == 0)
    def _():
        pltpu.sync_copy(den_sh, denp_h.at[c])
        pltpu.sync_copy(out_sh.at[pl.ds(NS * RPT, N - NS * RPT)],
                        outp_h.at[c, pl.ds(NS * RPT, N - NS * RPT)])


def _edges(u, v, xl, att1d, sdm):
    mesh = plsc.VectorSubcoreMesh(core_axis_name="c", subcore_axis_name="s")
    f = pl.kernel(
        _edges_body,
        out_type=[
            jax.ShapeDtypeStruct((NC, N, D), jnp.float32),
            jax.ShapeDtypeStruct((NC, N), jnp.float32),
        ],
        mesh=mesh,
        compiler_params=pltpu.CompilerParams(needs_layout_passes=False),
        scratch_types=[
            pltpu.VMEM((2, 2, B), jnp.int32),
            pltpu.VMEM((D,), jnp.float32),
            pltpu.VMEM((B, D), jnp.float32),
            pltpu.VMEM((B, D), jnp.float32),
            pltpu.VMEM((B, D), jnp.float32),
            pltpu.VMEM((B, 16), jnp.float32),
            pltpu.VMEM((2, B), jnp.float32),
            pltpu.VMEM((ZR, D), jnp.float32),
            pltpu.VMEM((1000,), jnp.float32),
            pltpu.VMEM_SHARED((N, D), jnp.float32),
            pltpu.VMEM_SHARED((N,), jnp.float32),
            pltpu.SemaphoreType.DMA,
            pltpu.SemaphoreType.DMA,
            pltpu.SemaphoreType.DMA,
            pltpu.SemaphoreType.DMA,
        ],
    )
    return f(u, v, xl, att1d, sdm)


def _finalize_body(outp_ref, denp_ref, gamma_ref, beta_ref, o_ref):
    p = outp_ref[0] + outp_ref[1]
    dsum = (denp_ref[0] + denp_ref[1] + 1e-16).reshape(N, 1)
    out = p / dsum
    mean = jnp.mean(out, axis=0, keepdims=True)
    var = jnp.mean((out - mean) ** 2, axis=0, keepdims=True)
    o_ref[...] = (out - mean) / jnp.sqrt(var + 1e-5) * gamma_ref[...] + beta_ref[...]


def _finalize(outp, denp, gamma, beta):
    return pl.pallas_call(
        _finalize_body,
        out_shape=jax.ShapeDtypeStruct((N, D), jnp.float32),
    )(outp, denp.reshape(NC, N, 1), gamma.reshape(1, D), beta.reshape(1, D))


def kernel(x, pos, edge_index, W_l, W_r, W_e, att, gamma, beta):
    sdm = jnp.stack([edge_index[0].reshape(NW, NCH, B),
                     edge_index[1].reshape(NW, NCH, B)],
                    axis=2)
    u, v, xl = _feats(x, pos, W_l, W_r, W_e, att.reshape(1, D))
    outp, denp = _edges(u, v, xl, att.reshape(D), sdm)
    return _finalize(outp, denp, gamma, beta)
